# Initial kernel scaffold; baseline (speedup 1.0000x reference)
import jax
import jax.numpy as jnp
from jax.experimental import pallas as pl


def kernel(x, edge_index, edge_feat, time_deltas, gammas, W_msg, b_msg, W_attn, b_attn, W_out, b_out, ln_w, ln_b):
    raise NotImplementedError("write your pallas kernel here")


# speedup vs baseline: 2.3971x; 2.3971x over previous
import dataclasses

import jax
import jax.numpy as jnp
from jax import lax
from jax.experimental import pallas as pl
from jax.experimental.pallas import tpu as pltpu
from jax.experimental.pallas import tpu_sc as plsc

N = 10000
E = 320000
NODE_DIM = 128
OUT_DIM = 128
N_HEADS = 4

def _sc_compiler_params():
    cp = pltpu.CompilerParams()
    if "needs_layout_passes" in pltpu.CompilerParams.__dataclass_fields__:
        cp = dataclasses.replace(cp, needs_layout_passes=False)
    return cp


NC = 2
NS = 16
NTILES = NC * NS
EPT = E // NTILES

CH2 = 2000
CH4 = 200

ROWS_MOST = 632
ROWS_LAST = N - 15 * ROWS_MOST


def _k1a_body(x_ref, w1_ref, wa1_ref, wa2_ref, we_ref, wt_ref, bm_ref,
              ba_ref, gam_ref, xw_ref, g_ref, acat_ref, c0_ref, sg_ref):
    x = x_ref[...]
    xw = jnp.dot(x, w1_ref[...], preferred_element_type=jnp.float32)
    xw_ref[...] = xw
    wa1 = wa1_ref[...]
    g1 = jnp.dot(xw, wa1, preferred_element_type=jnp.float32)
    g2 = jnp.dot(x, wa2_ref[...], preferred_element_type=jnp.float32)
    g_ref[...] = jnp.concatenate([g1, g2], axis=1)
    ae = jnp.dot(we_ref[...], wa1, preferred_element_type=jnp.float32)
    at = jnp.dot(wt_ref[...], wa1, preferred_element_type=jnp.float32)
    acat_ref[...] = jnp.concatenate([ae, at], axis=0)
    c0_ref[...] = (jnp.dot(bm_ref[...], wa1, preferred_element_type=jnp.float32)
                   + ba_ref[...])
    g = gam_ref[...]
    sg_ref[...] = jnp.maximum(g, 0.0) + jnp.log1p(jnp.exp(-jnp.abs(g)))


def _run_k1a(x, w1, wa1, wa2, we, wt, bm2, ba2, gam2):
    return pl.pallas_call(
        _k1a_body,
        out_shape=[
            jax.ShapeDtypeStruct((N, NODE_DIM), jnp.float32),
            jax.ShapeDtypeStruct((N, 8), jnp.float32),
            jax.ShapeDtypeStruct((32, N_HEADS), jnp.float32),
            jax.ShapeDtypeStruct((1, N_HEADS), jnp.float32),
            jax.ShapeDtypeStruct((1, 16), jnp.float32),
        ],
    )(x, w1, wa1, wa2, we, wt, bm2, ba2, gam2)


EB = 4000


def _k1b_body(dt_ref, ef_ref, acat_ref, sg_ref, c0_ref, ft_ref, loc_ref):
    dt = jnp.maximum(dt_ref[...], 0.0)
    te = jnp.exp(-(dt * sg_ref[...]))
    ft = jnp.concatenate([ef_ref[...], te], axis=1)
    ft_ref[...] = ft
    loc_ref[...] = (jnp.dot(ft, acat_ref[...], preferred_element_type=jnp.float32)
                    + c0_ref[...])


def _run_k1b(dt2, ef, acat, sg, c0):
    nblk = E // EB
    return pl.pallas_call(
        _k1b_body,
        grid=(nblk,),
        in_specs=[
            pl.BlockSpec((EB, 1), lambda i: (i, 0)),
            pl.BlockSpec((EB, 16), lambda i: (i, 0)),
            pl.BlockSpec((32, N_HEADS), lambda i: (0, 0)),
            pl.BlockSpec((1, 16), lambda i: (0, 0)),
            pl.BlockSpec((1, N_HEADS), lambda i: (0, 0)),
        ],
        out_specs=[
            pl.BlockSpec((EB, 32), lambda i: (i, 0)),
            pl.BlockSpec((EB, N_HEADS), lambda i: (i, 0)),
        ],
        out_shape=[
            jax.ShapeDtypeStruct((E, 32), jnp.float32),
            jax.ShapeDtypeStruct((E, N_HEADS), jnp.float32),
        ],
    )(dt2, ef, acat, sg, c0)


def _k2_body(g_hbm, src_hbm, dst_hbm, loc_hbm, l_hbm, gt, srcb, dstb, locb, lb):
    c = lax.axis_index("c")
    s = lax.axis_index("s")
    wid = c * NS + s
    base = wid * EPT
    pltpu.sync_copy(g_hbm, gt)
    iota16 = lax.iota(jnp.int32, 16)
    iota4 = iota16 * 4
    hvecs = [jnp.full((16,), h, jnp.int32) for h in range(2 * N_HEADS)]

    @pl.loop(0, EPT, step=CH2)
    def _chunk(e0):
        gb = base + e0
        pltpu.sync_copy(src_hbm.at[pl.ds(gb, CH2)], srcb)
        pltpu.sync_copy(dst_hbm.at[pl.ds(gb, CH2)], dstb)
        pltpu.sync_copy(loc_hbm.at[pl.ds(gb * N_HEADS, CH2 * N_HEADS)], locb)

        @pl.loop(0, CH2, step=16)
        def _vec(cc):
            sv8 = srcb[pl.ds(cc, 16)] * 8
            dv8 = dstb[pl.ds(cc, 16)] * 8
            lbase = iota4 + cc * 4
            for h in range(N_HEADS):
                g1 = plsc.load_gather(gt, [sv8 + hvecs[h]])
                g2 = plsc.load_gather(gt, [dv8 + hvecs[h + N_HEADS]])
                lo = plsc.load_gather(locb, [lbase + hvecs[h]])
                lb[pl.ds(h * CH2 + cc, 16)] = g1 + g2 + lo

        for h in range(N_HEADS):
            pltpu.sync_copy(lb.at[pl.ds(h * CH2, CH2)],
                            l_hbm.at[pl.ds(h * E + gb, CH2)])


def _run_k2(gflat, src, dst, locflat):
    mesh = plsc.VectorSubcoreMesh(core_axis_name="c", subcore_axis_name="s")
    f = pl.kernel(
        _k2_body,
        out_type=jax.ShapeDtypeStruct((N_HEADS * E,), jnp.float32),
        mesh=mesh,
        compiler_params=_sc_compiler_params(),
        scratch_types=[
            pltpu.VMEM((N * 8,), jnp.float32),
            pltpu.VMEM((CH2,), jnp.int32),
            pltpu.VMEM((CH2,), jnp.int32),
            pltpu.VMEM((CH2 * N_HEADS,), jnp.float32),
            pltpu.VMEM((N_HEADS * CH2,), jnp.float32),
        ],
    )
    return f(gflat, src, dst, locflat)


def _k3_body(l_ref, w_ref):
    l = l_ref[...]
    m = jnp.max(l, axis=(1, 2), keepdims=True)
    p = jnp.exp(l - m)
    z = jnp.sum(p, axis=(1, 2), keepdims=True)
    w_ref[...] = jnp.mean(p / z, axis=0)


def _run_k3(l3):
    return pl.pallas_call(
        _k3_body,
        out_shape=jax.ShapeDtypeStruct((E // 128, 128), jnp.float32),
    )(l3)


def _zero_rows(buf, nrows):
    zero16 = jnp.zeros((16,), jnp.float32)
    ncols = buf.shape[1]

    @pl.loop(0, nrows)
    def _z(i):
        for j in range(ncols // 16):
            buf[i, pl.ds(16 * j, 16)] = zero16


def _zero_shared(zbuf, sh, s):
    row0 = s * ROWS_MOST

    @pl.when(s < 15)
    def _zmost():
        for off, h in ((0, 200), (200, 200), (400, 200), (600, 32)):
            pltpu.sync_copy(zbuf.at[pl.ds(0, h)], sh.at[pl.ds(row0 + off, h)])

    @pl.when(s == 15)
    def _zlast():
        for off, h in ((0, 200), (200, 200), (400, 120)):
            pltpu.sync_copy(zbuf.at[pl.ds(0, h)], sh.at[pl.ds(row0 + off, h)])


def _copy_out_shared(sh, out_hbm, c, s):
    row0 = s * ROWS_MOST

    @pl.when(s < 15)
    def _omost():
        for off, h in ((0, 200), (200, 200), (400, 200), (600, 32)):
            pltpu.sync_copy(sh.at[pl.ds(row0 + off, h)],
                            out_hbm.at[c, pl.ds(row0 + off, h)])

    @pl.when(s == 15)
    def _olast():
        for off, h in ((0, 200), (200, 200), (400, 120)):
            pltpu.sync_copy(sh.at[pl.ds(row0 + off, h)],
                            out_hbm.at[c, pl.ds(row0 + off, h)])


def _k4a_body(xw_hbm, src_hbm, dst_hbm, w_hbm, ap_hbm,
              agg_sh, srcb, dstb, wb, r1):
    c = lax.axis_index("c")
    s = lax.axis_index("s")
    wid = c * NS + s

    _zero_rows(r1, 200)
    _zero_shared(r1, agg_sh, s)
    plsc.subcore_barrier()

    @pl.loop(0, EPT, step=CH4)
    def _chunk(e0):
        gb = wid * EPT + e0
        pltpu.sync_copy(src_hbm.at[pl.ds(gb, CH4)], srcb)
        pltpu.sync_copy(dst_hbm.at[pl.ds(gb, CH4)], dstb)
        pltpu.sync_copy(w_hbm.at[pl.ds(gb, CH4)], wb)
        pltpu.sync_copy(xw_hbm.at[srcb], r1)

        @pl.loop(0, CH4)
        def _edge(i):
            wv = plsc.load_gather(wb, [jnp.full((16,), i, jnp.int32)])
            for j in range(8):
                r1[i, pl.ds(16 * j, 16)] = r1[i, pl.ds(16 * j, 16)] * wv

        pltpu.sync_copy(r1, agg_sh.at[dstb], add=True)

    plsc.subcore_barrier()
    _copy_out_shared(agg_sh, ap_hbm, c, s)


def _run_k4a(xw, src, dst, w):
    mesh = plsc.VectorSubcoreMesh(core_axis_name="c", subcore_axis_name="s")
    f = pl.kernel(
        _k4a_body,
        out_type=jax.ShapeDtypeStruct((NC, N, NODE_DIM), jnp.float32),
        mesh=mesh,
        compiler_params=_sc_compiler_params(),
        scratch_types=[
            pltpu.VMEM_SHARED((N, NODE_DIM), jnp.float32),
            pltpu.VMEM((CH4,), jnp.int32),
            pltpu.VMEM((CH4,), jnp.int32),
            pltpu.VMEM((CH4,), jnp.float32),
            pltpu.VMEM((CH4, NODE_DIM), jnp.float32),
        ],
    )
    return f(xw, src, dst, w)


def _k4b_body(dst_hbm, w_hbm, ft_hbm, cp_hbm, c_sh, dstb, wb, ftb, r2):
    c = lax.axis_index("c")
    s = lax.axis_index("s")
    wid = c * NS + s
    onehot0 = jnp.where(lax.iota(jnp.int32, 16) == 0, 1.0, 0.0)

    _zero_rows(r2, 200)
    _zero_shared(r2, c_sh, s)
    plsc.subcore_barrier()

    @pl.loop(0, EPT, step=CH4)
    def _chunk(e0):
        gb = wid * EPT + e0
        pltpu.sync_copy(dst_hbm.at[pl.ds(gb, CH4)], dstb)
        pltpu.sync_copy(w_hbm.at[pl.ds(gb, CH4)], wb)
        pltpu.sync_copy(ft_hbm.at[pl.ds(gb * 32, CH4 * 32)], ftb)

        @pl.loop(0, CH4)
        def _edge(i):
            wv = plsc.load_gather(wb, [jnp.full((16,), i, jnp.int32)])
            fb = i * 32
            r2[i, pl.ds(0, 16)] = ftb[pl.ds(fb, 16)] * wv
            r2[i, pl.ds(16, 16)] = ftb[pl.ds(fb + 16, 16)] * wv
            r2[i, pl.ds(32, 16)] = onehot0 * wv

        pltpu.sync_copy(r2, c_sh.at[dstb], add=True)

    plsc.subcore_barrier()
    _copy_out_shared(c_sh, cp_hbm, c, s)


def _run_k4b(dst, w, ftflat):
    mesh = plsc.VectorSubcoreMesh(core_axis_name="c", subcore_axis_name="s")
    f = pl.kernel(
        _k4b_body,
        out_type=jax.ShapeDtypeStruct((NC, N, 128), jnp.float32),
        mesh=mesh,
        compiler_params=_sc_compiler_params(),
        scratch_types=[
            pltpu.VMEM_SHARED((N, 128), jnp.float32),
            pltpu.VMEM((CH4,), jnp.int32),
            pltpu.VMEM((CH4,), jnp.float32),
            pltpu.VMEM((CH4 * 32,), jnp.float32),
            pltpu.VMEM((CH4, 128), jnp.float32),
        ],
    )
    return f(dst, w, ftflat)


def _k5_body(ap_ref, cp_ref, x_ref, we_ref, wt_ref, bm_ref, woa_ref, wox_ref,
             bo_ref, lnw_ref, lnb_ref, out_ref):
    agg = ap_ref[0] + ap_ref[1]
    cc = cp_ref[0] + cp_ref[1]
    f = cc[:, 0:16]
    t = cc[:, 16:32]
    sw = cc[:, 32:33]
    agg = (agg
           + jnp.dot(f, we_ref[...], preferred_element_type=jnp.float32)
           + jnp.dot(t, wt_ref[...], preferred_element_type=jnp.float32)
           + sw * bm_ref[...])
    h = (jnp.dot(agg, woa_ref[...], preferred_element_type=jnp.float32)
         + jnp.dot(x_ref[...], wox_ref[...], preferred_element_type=jnp.float32)
         + bo_ref[...])
    h = 0.5 * h * (1.0 + lax.erf(h * 0.7071067811865476))
    mu = jnp.mean(h, axis=-1, keepdims=True)
    hc = h - mu
    var = jnp.mean(hc * hc, axis=-1, keepdims=True)
    out_ref[...] = hc * lax.rsqrt(var + 1e-5) * lnw_ref[...] + lnb_ref[...]


def _run_k5(ap, cp, x, we, wt, bm2, woa, wox, bo2, lnw2, lnb2):
    return pl.pallas_call(
        _k5_body,
        out_shape=jax.ShapeDtypeStruct((N, OUT_DIM), jnp.float32),
    )(ap, cp, x, we, wt, bm2, woa, wox, bo2, lnw2, lnb2)


def kernel(x, edge_index, edge_feat, time_deltas, gammas, W_msg, b_msg,
           W_attn, b_attn, W_out, b_out, ln_w, ln_b):
    src = edge_index[0]
    dst = edge_index[1]
    w1 = W_msg[:NODE_DIM]
    we = W_msg[NODE_DIM:NODE_DIM + 16]
    wt = W_msg[NODE_DIM + 16:]
    wa1 = W_attn[:OUT_DIM]
    wa2 = W_attn[OUT_DIM:]

    xw, g, acat, c0, sg = _run_k1a(
        x, w1, wa1, wa2, we, wt, b_msg.reshape(1, OUT_DIM),
        b_attn.reshape(1, N_HEADS), gammas.reshape(1, 16))

    ft, loc = _run_k1b(time_deltas.reshape(E, 1), edge_feat, acat, sg, c0)

    logits = _run_k2(g.reshape(-1), src, dst, loc.reshape(-1))

    w = _run_k3(logits.reshape(N_HEADS, E // 128, 128)).reshape(E)

    ap = _run_k4a(xw, src, dst, w)
    cp = _run_k4b(dst, w, ft.reshape(-1))

    return _run_k5(ap, cp, x, we, wt, b_msg.reshape(1, OUT_DIM),
                   W_out[:OUT_DIM], W_out[OUT_DIM:],
                   b_out.reshape(1, OUT_DIM), ln_w.reshape(1, OUT_DIM),
                   ln_b.reshape(1, OUT_DIM))
